# Initial kernel scaffold; baseline (speedup 1.0000x reference)
import jax
import jax.numpy as jnp
from jax.experimental import pallas as pl


def kernel(nodes, neigh_idx, features, weight):
    raise NotImplementedError("write your pallas kernel here")


# speedup vs baseline: 1.9522x; 1.9522x over previous
import functools

import jax
import jax.numpy as jnp
from jax import lax
from jax.experimental import pallas as pl
from jax.experimental.pallas import tpu as pltpu
from jax.experimental.pallas import tpu_sc as plsc

D = 128
EMB = 128
B = 16384
S = 10

NC = 2
NS = 16
NW = NC * NS
ROWS_W = B // NW
CB = 64
NCHUNK = ROWS_W // CB
GR = CB * S
GSTEP = 128
SB = 128
NSCHUNK = ROWS_W // SB


def _sc_body(nodes_hbm, neigh_hbm, feat_hbm, self_hbm, sum_hbm,
             nidx_v, rows_v, acc_v, sidx_v, srows_v, sem):
    wid = lax.axis_index("s") * NC + lax.axis_index("c")
    base = wid * ROWS_W

    for c in range(NSCHUNK):
        row0 = base + c * SB
        pltpu.sync_copy(nodes_hbm.at[pl.ds(row0, SB)], sidx_v)
        pltpu.async_copy(feat_hbm.at[sidx_v], srows_v, sem).wait()
        pltpu.sync_copy(srows_v, self_hbm.at[pl.ds(row0, SB)])

    for c in range(NCHUNK):
        row0 = base + c * CB
        pltpu.sync_copy(neigh_hbm.at[pl.ds(row0 * S, GR)], nidx_v)
        for j in range(GR // GSTEP):
            pltpu.async_copy(
                feat_hbm.at[nidx_v.at[pl.ds(j * GSTEP, GSTEP)]],
                rows_v.at[pl.ds(j * GSTEP, GSTEP)], sem).wait()

        @pl.loop(0, CB)
        def _acc(b):
            r0 = b * S
            for dblk in range(D // 16):
                dsl = pl.ds(dblk * 16, 16)
                v = rows_v[r0, dsl]
                for s in range(1, S):
                    v = v + rows_v[r0 + s, dsl]
                acc_v[b, dsl] = v

        pltpu.sync_copy(acc_v, sum_hbm.at[pl.ds(row0, CB)])


_sc_gather = pl.kernel(
    _sc_body,
    out_type=[jax.ShapeDtypeStruct((B, D), jnp.float32),
              jax.ShapeDtypeStruct((B, D), jnp.float32)],
    mesh=plsc.VectorSubcoreMesh(core_axis_name="c", subcore_axis_name="s"),
    scratch_types=[
        pltpu.VMEM((GR,), jnp.int32),
        pltpu.VMEM((GR, D), jnp.float32),
        pltpu.VMEM((CB, D), jnp.float32),
        pltpu.VMEM((SB,), jnp.int32),
        pltpu.VMEM((SB, D), jnp.float32),
        pltpu.SemaphoreType.DMA,
    ],
)

BT = 2048


def _tc_body(w_ref, self_ref, sum_ref, out_ref):
    w1 = w_ref[:, :D]
    w2 = w_ref[:, D:]
    a = lax.dot_general(w1, self_ref[...], (((1,), (1,)), ((), ())),
                        preferred_element_type=jnp.float32)
    b = lax.dot_general(w2, sum_ref[...], (((1,), (1,)), ((), ())),
                        preferred_element_type=jnp.float32)
    out_ref[...] = jnp.maximum(a + b * (1.0 / S), 0.0)


def _tc_matmul(weight, self_rows, neigh_sum):
    return pl.pallas_call(
        _tc_body,
        grid=(B // BT,),
        in_specs=[
            pl.BlockSpec((EMB, 2 * D), lambda i: (0, 0)),
            pl.BlockSpec((BT, D), lambda i: (i, 0)),
            pl.BlockSpec((BT, D), lambda i: (i, 0)),
        ],
        out_specs=pl.BlockSpec((EMB, BT), lambda i: (0, i)),
        out_shape=jax.ShapeDtypeStruct((EMB, B), jnp.float32),
    )(weight, self_rows, neigh_sum)


def kernel(nodes, neigh_idx, features, weight):
    neigh_flat = neigh_idx.reshape(-1)
    self_rows, neigh_sum = _sc_gather(nodes, neigh_flat, features)
    return _tc_matmul(weight, self_rows, neigh_sum)


# speedup vs baseline: 2.8549x; 1.4624x over previous
import functools

import jax
import jax.numpy as jnp
from jax import lax
from jax.experimental import pallas as pl
from jax.experimental.pallas import tpu as pltpu
from jax.experimental.pallas import tpu_sc as plsc

D = 128
EMB = 128
B = 16384
S = 10

NC = 2
NS = 16
NW = NC * NS
ROWS_W = B // NW
CB = 32
NCHUNK = ROWS_W // CB
GR = CB * S
GSLICES = [(0, 128), (128, 128), (256, 64)]


def _sc_body(nodes_hbm, neigh_hbm, feat_hbm, self_hbm, sum_hbm,
             nidx_v, sidx_v, rows0_v, rows1_v, acc0_v, acc1_v,
             sbuf0_v, sbuf1_v, sem0, sem1):
    wid = lax.axis_index("s") * NC + lax.axis_index("c")
    base = wid * ROWS_W
    rows = (rows0_v, rows1_v)
    accs = (acc0_v, acc1_v)
    sbufs = (sbuf0_v, sbuf1_v)
    sems = (sem0, sem1)

    pltpu.sync_copy(neigh_hbm.at[pl.ds(base * S, ROWS_W * S)], nidx_v)
    pltpu.sync_copy(nodes_hbm.at[pl.ds(base, ROWS_W)], sidx_v)

    def start_gathers(c, p):
        descs = []
        for (off, ln) in GSLICES:
            descs.append(pltpu.async_copy(
                feat_hbm.at[nidx_v.at[pl.ds(c * GR + off, ln)]],
                rows[p].at[pl.ds(off, ln)], sems[p]))
        descs.append(pltpu.async_copy(
            feat_hbm.at[sidx_v.at[pl.ds(c * CB, CB)]], sbufs[p], sems[p]))
        return descs

    inflight = start_gathers(0, 0)
    for c in range(NCHUNK):
        p = c % 2
        cur = inflight
        if c + 1 < NCHUNK:
            inflight = start_gathers(c + 1, 1 - p)
        for d in cur:
            d.wait()
        pltpu.sync_copy(sbufs[p], self_hbm.at[pl.ds(base + c * CB, CB)])

        rv = rows[p]
        av = accs[p]

        @pl.loop(0, CB)
        def _acc(b):
            r0 = b * S
            for dblk in range(D // 16):
                dsl = pl.ds(dblk * 16, 16)
                v = rv[r0, dsl]
                for s in range(1, S):
                    v = v + rv[r0 + s, dsl]
                av[b, dsl] = v

        pltpu.sync_copy(av, sum_hbm.at[pl.ds(base + c * CB, CB)])


_sc_gather = pl.kernel(
    _sc_body,
    out_type=[jax.ShapeDtypeStruct((B, D), jnp.float32),
              jax.ShapeDtypeStruct((B, D), jnp.float32)],
    mesh=plsc.VectorSubcoreMesh(core_axis_name="c", subcore_axis_name="s"),
    scratch_types=[
        pltpu.VMEM((ROWS_W * S,), jnp.int32),
        pltpu.VMEM((ROWS_W,), jnp.int32),
        pltpu.VMEM((GR, D), jnp.float32),
        pltpu.VMEM((GR, D), jnp.float32),
        pltpu.VMEM((CB, D), jnp.float32),
        pltpu.VMEM((CB, D), jnp.float32),
        pltpu.VMEM((CB, D), jnp.float32),
        pltpu.VMEM((CB, D), jnp.float32),
        pltpu.SemaphoreType.DMA,
        pltpu.SemaphoreType.DMA,
    ],
)

BT = 2048


def _tc_body(w_ref, self_ref, sum_ref, out_ref):
    w1 = w_ref[:, :D]
    w2 = w_ref[:, D:]
    a = lax.dot_general(w1, self_ref[...], (((1,), (1,)), ((), ())),
                        preferred_element_type=jnp.float32)
    b = lax.dot_general(w2, sum_ref[...], (((1,), (1,)), ((), ())),
                        preferred_element_type=jnp.float32)
    out_ref[...] = jnp.maximum(a + b * (1.0 / S), 0.0)


def _tc_matmul(weight, self_rows, neigh_sum):
    return pl.pallas_call(
        _tc_body,
        grid=(B // BT,),
        in_specs=[
            pl.BlockSpec((EMB, 2 * D), lambda i: (0, 0)),
            pl.BlockSpec((BT, D), lambda i: (i, 0)),
            pl.BlockSpec((BT, D), lambda i: (i, 0)),
        ],
        out_specs=pl.BlockSpec((EMB, BT), lambda i: (0, i)),
        out_shape=jax.ShapeDtypeStruct((EMB, B), jnp.float32),
    )(weight, self_rows, neigh_sum)


def kernel(nodes, neigh_idx, features, weight):
    neigh_flat = neigh_idx.reshape(-1)
    self_rows, neigh_sum = _sc_gather(nodes, neigh_flat, features)
    return _tc_matmul(weight, self_rows, neigh_sum)


# speedup vs baseline: 3.2813x; 1.1494x over previous
import functools

import jax
import jax.numpy as jnp
from jax import lax
from jax.experimental import pallas as pl
from jax.experimental.pallas import tpu as pltpu
from jax.experimental.pallas import tpu_sc as plsc

D = 128
EMB = 128
B = 16384
S = 10

NC = 2
NS = 16
NW = NC * NS
ROWS_W = B // NW
CB = 32
NCHUNK = ROWS_W // CB


def _sc_body(nodes_hbm, neigh_hbm, feat_hbm, self_hbm, sum_hbm,
             nidx_v, sidx_v, rows0_v, rows1_v, acc0_v, acc1_v,
             sbuf0_v, sbuf1_v, semg0, semg1, semso0, semso1, semao0, semao1):
    wid = lax.axis_index("s") * NC + lax.axis_index("c")
    base = wid * ROWS_W
    rows = (rows0_v, rows1_v)
    accs = (acc0_v, acc1_v)
    sbufs = (sbuf0_v, sbuf1_v)
    semg = (semg0, semg1)
    semso = (semso0, semso1)
    semao = (semao0, semao1)

    pltpu.sync_copy(neigh_hbm.at[pl.ds(base * S, ROWS_W * S)], nidx_v)
    pltpu.sync_copy(nodes_hbm.at[pl.ds(base, ROWS_W)], sidx_v)

    def start_gathers(c, p):
        descs = []
        for (off, ln) in [(0, 128), (128, 128), (256, 64)]:
            descs.append(pltpu.async_copy(
                feat_hbm.at[nidx_v.at[pl.ds(c * CB * S + off, ln)]],
                rows[p].at[pl.ds(off, ln)], semg[p]))
        descs.append(pltpu.async_copy(
            feat_hbm.at[sidx_v.at[pl.ds(c * CB, CB)]], sbufs[p], semg[p]))
        return descs

    pending = {}

    def drain(slot):
        d = pending.pop(slot, None)
        if d is not None:
            d.wait()

    inflight = start_gathers(0, 0)
    for c in range(NCHUNK):
        p = c % 2
        cur = inflight
        if c + 1 < NCHUNK:
            drain(("so", 1 - p))
            inflight = start_gathers(c + 1, 1 - p)
        for d in cur:
            d.wait()
        pending[("so", p)] = pltpu.async_copy(
            sbufs[p], self_hbm.at[pl.ds(base + c * CB, CB)], semso[p])
        drain(("ao", p))

        rv = rows[p]
        av = accs[p]

        @pl.loop(0, CB)
        def _acc(b):
            r0 = b * S
            for dblk in range(D // 16):
                dsl = pl.ds(dblk * 16, 16)
                vs = [rv[r0 + s, dsl] for s in range(S)]
                while len(vs) > 1:
                    nxt = [vs[i] + vs[i + 1] for i in range(0, len(vs) - 1, 2)]
                    if len(vs) % 2:
                        nxt.append(vs[-1])
                    vs = nxt
                av[b, dsl] = vs[0]

        pending[("ao", p)] = pltpu.async_copy(
            accs[p], sum_hbm.at[pl.ds(base + c * CB, CB)], semao[p])

    for slot in [("so", 0), ("so", 1), ("ao", 0), ("ao", 1)]:
        drain(slot)


_sc_gather = pl.kernel(
    _sc_body,
    out_type=[jax.ShapeDtypeStruct((B, D), jnp.float32),
              jax.ShapeDtypeStruct((B, D), jnp.float32)],
    mesh=plsc.VectorSubcoreMesh(core_axis_name="c", subcore_axis_name="s"),
    scratch_types=[
        pltpu.VMEM((ROWS_W * S,), jnp.int32),
        pltpu.VMEM((ROWS_W,), jnp.int32),
        pltpu.VMEM((CB * S, D), jnp.float32),
        pltpu.VMEM((CB * S, D), jnp.float32),
        pltpu.VMEM((CB, D), jnp.float32),
        pltpu.VMEM((CB, D), jnp.float32),
        pltpu.VMEM((CB, D), jnp.float32),
        pltpu.VMEM((CB, D), jnp.float32),
        pltpu.SemaphoreType.DMA,
        pltpu.SemaphoreType.DMA,
        pltpu.SemaphoreType.DMA,
        pltpu.SemaphoreType.DMA,
        pltpu.SemaphoreType.DMA,
        pltpu.SemaphoreType.DMA,
    ],
)

BT = 2048


def _tc_body(w_ref, self_ref, sum_ref, out_ref):
    w1 = w_ref[:, :D]
    w2 = w_ref[:, D:]
    a = lax.dot_general(w1, self_ref[...], (((1,), (1,)), ((), ())),
                        preferred_element_type=jnp.float32)
    b = lax.dot_general(w2, sum_ref[...], (((1,), (1,)), ((), ())),
                        preferred_element_type=jnp.float32)
    out_ref[...] = jnp.maximum(a + b * (1.0 / S), 0.0)


def _tc_matmul(weight, self_rows, neigh_sum):
    return pl.pallas_call(
        _tc_body,
        grid=(B // BT,),
        in_specs=[
            pl.BlockSpec((EMB, 2 * D), lambda i: (0, 0)),
            pl.BlockSpec((BT, D), lambda i: (i, 0)),
            pl.BlockSpec((BT, D), lambda i: (i, 0)),
        ],
        out_specs=pl.BlockSpec((EMB, BT), lambda i: (0, i)),
        out_shape=jax.ShapeDtypeStruct((EMB, B), jnp.float32),
    )(weight, self_rows, neigh_sum)


def kernel(nodes, neigh_idx, features, weight):
    neigh_flat = neigh_idx.reshape(-1)
    self_rows, neigh_sum = _sc_gather(nodes, neigh_flat, features)
    return _tc_matmul(weight, self_rows, neigh_sum)


# speedup vs baseline: 3.7945x; 1.1564x over previous
import functools

import jax
import jax.numpy as jnp
from jax import lax
from jax.experimental import pallas as pl
from jax.experimental.pallas import tpu as pltpu
from jax.experimental.pallas import tpu_sc as plsc

D = 128
EMB = 128
B = 16384
S = 10

NC = 2
NS = 16
NW = NC * NS
ROWS_W = B // NW
CB = 32
NCHUNK = ROWS_W // CB


def _sc_body(nodes_hbm, neigh_hbm, feat_hbm, self_hbm, sum_hbm,
             nidx_v, sidx_v, rows0_v, rows1_v, acc0_v, acc1_v,
             sbuf0_v, sbuf1_v, semg0, semg1, semso0, semso1, semao0, semao1):
    wid = lax.axis_index("s") * NC + lax.axis_index("c")
    base = wid * ROWS_W
    rows = (rows0_v, rows1_v)
    accs = (acc0_v, acc1_v)
    sbufs = (sbuf0_v, sbuf1_v)
    semg = (semg0, semg1)
    semso = (semso0, semso1)
    semao = (semao0, semao1)

    pltpu.sync_copy(neigh_hbm.at[pl.ds(base * S, ROWS_W * S)], nidx_v)
    pltpu.sync_copy(nodes_hbm.at[pl.ds(base, ROWS_W)], sidx_v)

    def start_gathers(c, p):
        descs = []
        for (off, ln) in [(0, 128), (128, 128), (256, 64)]:
            descs.append(pltpu.async_copy(
                feat_hbm.at[nidx_v.at[pl.ds(c * CB * S + off, ln)]],
                rows[p].at[pl.ds(off, ln)], semg[p]))
        descs.append(pltpu.async_copy(
            feat_hbm.at[sidx_v.at[pl.ds(c * CB, CB)]], sbufs[p], semg[p]))
        return descs

    pending = {}

    def drain(slot):
        d = pending.pop(slot, None)
        if d is not None:
            d.wait()

    inflight = start_gathers(0, 0)
    for c in range(NCHUNK):
        p = c % 2
        cur = inflight
        if c + 1 < NCHUNK:
            drain(("so", 1 - p))
            inflight = start_gathers(c + 1, 1 - p)
        for d in cur:
            d.wait()
        pending[("so", p)] = pltpu.async_copy(
            sbufs[p], self_hbm.at[pl.ds(base + c * CB, CB)], semso[p])
        drain(("ao", p))

        rv = rows[p]
        av = accs[p]

        @plsc.parallel_loop(0, CB)
        def _acc(b):
            r0 = b * S
            for g in range(D // 16 // 2):
                dsls = [pl.ds((2 * g + h) * 16, 16) for h in range(2)]
                lanes = [[], []]
                for s in range(S):
                    for h in range(2):
                        lanes[h].append(rv[r0 + s, dsls[h]])
                while len(lanes[0]) > 1:
                    nxt = [[], []]
                    for i in range(0, len(lanes[0]) - 1, 2):
                        for h in range(2):
                            nxt[h].append(lanes[h][i] + lanes[h][i + 1])
                    if len(lanes[0]) % 2:
                        for h in range(2):
                            nxt[h].append(lanes[h][-1])
                    lanes = nxt
                for h in range(2):
                    av[b, dsls[h]] = lanes[h][0]

        pending[("ao", p)] = pltpu.async_copy(
            accs[p], sum_hbm.at[pl.ds(base + c * CB, CB)], semao[p])

    for slot in [("so", 0), ("so", 1), ("ao", 0), ("ao", 1)]:
        drain(slot)


_sc_gather = pl.kernel(
    _sc_body,
    out_type=[jax.ShapeDtypeStruct((B, D), jnp.float32),
              jax.ShapeDtypeStruct((B, D), jnp.float32)],
    mesh=plsc.VectorSubcoreMesh(core_axis_name="c", subcore_axis_name="s"),
    scratch_types=[
        pltpu.VMEM((ROWS_W * S,), jnp.int32),
        pltpu.VMEM((ROWS_W,), jnp.int32),
        pltpu.VMEM((CB * S, D), jnp.float32),
        pltpu.VMEM((CB * S, D), jnp.float32),
        pltpu.VMEM((CB, D), jnp.float32),
        pltpu.VMEM((CB, D), jnp.float32),
        pltpu.VMEM((CB, D), jnp.float32),
        pltpu.VMEM((CB, D), jnp.float32),
        pltpu.SemaphoreType.DMA,
        pltpu.SemaphoreType.DMA,
        pltpu.SemaphoreType.DMA,
        pltpu.SemaphoreType.DMA,
        pltpu.SemaphoreType.DMA,
        pltpu.SemaphoreType.DMA,
    ],
)

BT = 2048


def _tc_body(w_ref, self_ref, sum_ref, out_ref):
    w1 = w_ref[:, :D]
    w2 = w_ref[:, D:]
    a = lax.dot_general(w1, self_ref[...], (((1,), (1,)), ((), ())),
                        preferred_element_type=jnp.float32)
    b = lax.dot_general(w2, sum_ref[...], (((1,), (1,)), ((), ())),
                        preferred_element_type=jnp.float32)
    out_ref[...] = jnp.maximum(a + b * (1.0 / S), 0.0)


def _tc_matmul(weight, self_rows, neigh_sum):
    return pl.pallas_call(
        _tc_body,
        grid=(B // BT,),
        in_specs=[
            pl.BlockSpec((EMB, 2 * D), lambda i: (0, 0)),
            pl.BlockSpec((BT, D), lambda i: (i, 0)),
            pl.BlockSpec((BT, D), lambda i: (i, 0)),
        ],
        out_specs=pl.BlockSpec((EMB, BT), lambda i: (0, i)),
        out_shape=jax.ShapeDtypeStruct((EMB, B), jnp.float32),
    )(weight, self_rows, neigh_sum)


def kernel(nodes, neigh_idx, features, weight):
    neigh_flat = neigh_idx.reshape(-1)
    self_rows, neigh_sum = _sc_gather(nodes, neigh_flat, features)
    return _tc_matmul(weight, self_rows, neigh_sum)
